# Initial kernel scaffold; baseline (speedup 1.0000x reference)
import functools

import jax
import jax.numpy as jnp
from jax import lax
from jax.experimental import pallas as pl
from jax.experimental.pallas import tpu as pltpu
from jax.experimental.pallas import tpu_sc as plsc

_K = 8192
_C = 256
_BETA = 0.25

_BN = 512
_BK = 2048


def _dist_argmin_body(z_ref, zz_ref, cb_ref, cc_ref, idx_ref, loss_ref,
                      minv_ref, mini_ref, acc_ref):
    n = pl.program_id(0)
    k = pl.program_id(1)
    nn = pl.num_programs(0)
    nk = pl.num_programs(1)

    @pl.when(k == 0)
    def _init():
        minv_ref[...] = jnp.full((_BN, 1), jnp.inf, jnp.float32)
        mini_ref[...] = jnp.zeros((_BN, 1), jnp.int32)

    dot = lax.dot_general(z_ref[...], cb_ref[...], (((1,), (1,)), ((), ())),
                          preferred_element_type=jnp.float32)
    d = (zz_ref[...] - 2.0 * dot) + cc_ref[...]
    lm = jnp.min(d, axis=1, keepdims=True)
    cols = lax.broadcasted_iota(jnp.int32, (_BN, _BK), 1)
    la = jnp.min(jnp.where(d == lm, cols, _K), axis=1, keepdims=True) + k * _BK
    better = lm < minv_ref[...]
    minv_ref[...] = jnp.where(better, lm, minv_ref[...])
    mini_ref[...] = jnp.where(better, la, mini_ref[...])

    @pl.when(k == nk - 1)
    def _finalize():
        idx_ref[...] = mini_ref[...]

        @pl.when(n == 0)
        def _zero():
            acc_ref[0, 0] = 0.0

        acc_ref[0, 0] += jnp.sum(minv_ref[...])

        @pl.when(n == nn - 1)
        def _emit_loss():
            loss_ref[0, 0] = acc_ref[0, 0]


def _dist_argmin(z, zz, codebook, cc):
    n_rows = z.shape[0]
    grid = (n_rows // _BN, _K // _BK)
    idx2, loss2 = pl.pallas_call(
        _dist_argmin_body,
        grid=grid,
        in_specs=[
            pl.BlockSpec((_BN, _C), lambda n, k: (n, 0)),
            pl.BlockSpec((_BN, 1), lambda n, k: (n, 0)),
            pl.BlockSpec((_BK, _C), lambda n, k: (k, 0)),
            pl.BlockSpec((1, _BK), lambda n, k: (0, k)),
        ],
        out_specs=[
            pl.BlockSpec((_BN, 1), lambda n, k: (n, 0)),
            pl.BlockSpec((1, 1), lambda n, k: (0, 0)),
        ],
        out_shape=[
            jax.ShapeDtypeStruct((n_rows, 1), jnp.int32),
            jax.ShapeDtypeStruct((1, 1), jnp.float32),
        ],
        scratch_shapes=[
            pltpu.VMEM((_BN, 1), jnp.float32),
            pltpu.VMEM((_BN, 1), jnp.int32),
            pltpu.SMEM((1, 1), jnp.float32),
        ],
        compiler_params=pltpu.CompilerParams(
            dimension_semantics=("arbitrary", "arbitrary")),
    )(z, zz, codebook, cc)
    return idx2.reshape(-1), loss2[0, 0]


def _sc_gather(table, idx):
    info = plsc.get_sparse_core_info()
    nw = info.num_cores * info.num_subcores
    n_rows = idx.shape[0]
    ch = 128
    b_per_w = n_rows // nw
    nch = b_per_w // ch
    idx3 = idx.reshape(nw, nch, ch)
    mesh = plsc.VectorSubcoreMesh(core_axis_name="c", subcore_axis_name="s")

    @functools.partial(
        pl.kernel,
        out_type=jax.ShapeDtypeStruct((n_rows, _C), jnp.float32),
        mesh=mesh,
        scratch_types=[
            pltpu.VMEM((nch, ch), jnp.int32),
            pltpu.VMEM((ch, _C), jnp.float32),
            pltpu.SemaphoreType.DMA,
        ],
    )
    def gather_kernel(table_hbm, idx_hbm, out_hbm, idx_v, rows_v, sem):
        wid = lax.axis_index("s") * info.num_cores + lax.axis_index("c")
        base = wid * b_per_w
        pltpu.sync_copy(idx_hbm.at[wid], idx_v)
        for j in range(nch):
            pltpu.async_copy(table_hbm.at[idx_v.at[j]], rows_v, sem).wait()
            pltpu.sync_copy(rows_v, out_hbm.at[pl.ds(base + j * ch, ch)])

    return gather_kernel(table, idx3)


def kernel(x, codebook):
    b, c, t, h, w = x.shape
    z = jnp.transpose(x, (0, 2, 3, 4, 1)).reshape(-1, c)
    zz = jnp.sum(z * z, axis=1, keepdims=True)
    cc = jnp.sum(codebook * codebook, axis=1)[None, :]
    idx, loss_sum = _dist_argmin(z, zz, codebook, cc)
    q = _sc_gather(codebook, idx)
    n_rows = z.shape[0]
    loss = loss_sum * ((1.0 + _BETA) / (n_rows * c))
    q_st = z + (q - z)
    x_q = jnp.transpose(q_st.reshape(b, t, h, w, c), (0, 4, 1, 2, 3))
    indices = idx.reshape(b, t, h, w)
    return x_q, loss, indices


# speedup vs baseline: 1.0419x; 1.0419x over previous
import functools

import jax
import jax.numpy as jnp
from jax import lax
from jax.experimental import pallas as pl
from jax.experimental.pallas import tpu as pltpu
from jax.experimental.pallas import tpu_sc as plsc

_K = 8192
_C = 256
_BETA = 0.25

_BN = 512
_BK = 2736
_NKW = 3


def _dist_argmin_body(z_ref, zz_ref, cb_ref, cc_ref, idx_ref, loss_ref,
                      accv_ref, acci_ref, lossv_ref, acc_ref):
    n = pl.program_id(0)
    k = pl.program_id(1)
    nn = pl.num_programs(0)
    nk = pl.num_programs(1)

    @pl.when(k == 0)
    def _init():
        accv_ref[...] = jnp.full((_BN, 1), jnp.inf, jnp.float32)
        acci_ref[...] = jnp.zeros((_BN, 1), jnp.int32)
        lossv_ref[...] = jnp.zeros((_BN, 1), jnp.float32)

    dot = lax.dot_general(z_ref[...], cb_ref[...], (((1,), (1,)), ((), ())),
                          preferred_element_type=jnp.float32)
    d = (zz_ref[...] - 2.0 * dot) + cc_ref[0]
    cols = lax.broadcasted_iota(jnp.int32, (_BN, _BK), 1) + k * _BK
    d = jnp.where(cols < _K, d, jnp.inf)
    wv = jnp.min(d, axis=1, keepdims=True)
    wi = jnp.min(jnp.where(d == wv, cols, _K), axis=1, keepdims=True)
    take = wv < accv_ref[...]
    accv_ref[...] = jnp.where(
        take, wv.astype(jnp.bfloat16).astype(jnp.float32), accv_ref[...])
    acci_ref[...] = jnp.where(take, wi, acci_ref[...])
    lossv_ref[...] = jnp.where(take, wv, lossv_ref[...])

    @pl.when(k == nk - 1)
    def _finalize():
        idx_ref[...] = acci_ref[...]

        @pl.when(n == 0)
        def _zero():
            acc_ref[0, 0] = 0.0

        acc_ref[0, 0] += jnp.sum(lossv_ref[...])

        @pl.when(n == nn - 1)
        def _emit_loss():
            loss_ref[0, 0] = acc_ref[0, 0]


def _dist_argmin(z_bf, zz, cb_bf, cc):
    n_rows = z_bf.shape[0]
    grid = (n_rows // _BN, _NKW)
    idx2, loss2 = pl.pallas_call(
        _dist_argmin_body,
        grid=grid,
        in_specs=[
            pl.BlockSpec((_BN, _C), lambda n, k: (n, 0)),
            pl.BlockSpec((_BN, 1), lambda n, k: (n, 0)),
            pl.BlockSpec((_BK, _C), lambda n, k: (k, 0)),
            pl.BlockSpec((1, 1, _BK), lambda n, k: (k, 0, 0)),
        ],
        out_specs=[
            pl.BlockSpec((_BN, 1), lambda n, k: (n, 0)),
            pl.BlockSpec(memory_space=pltpu.SMEM),
        ],
        out_shape=[
            jax.ShapeDtypeStruct((n_rows, 1), jnp.int32),
            jax.ShapeDtypeStruct((1, 1), jnp.float32),
        ],
        scratch_shapes=[
            pltpu.VMEM((_BN, 1), jnp.float32),
            pltpu.VMEM((_BN, 1), jnp.int32),
            pltpu.VMEM((_BN, 1), jnp.float32),
            pltpu.SMEM((1, 1), jnp.float32),
        ],
        compiler_params=pltpu.CompilerParams(
            dimension_semantics=("arbitrary", "arbitrary")),
    )(z_bf, zz, cb_bf, cc)
    return idx2.reshape(-1), loss2[0, 0]


def _sc_gather(table, idx):
    info = plsc.get_sparse_core_info()
    nw = info.num_cores * info.num_subcores
    n_rows = idx.shape[0]
    ch = 128
    b_per_w = n_rows // nw
    nch = b_per_w // ch
    idx3 = idx.reshape(nw, nch, ch)
    mesh = plsc.VectorSubcoreMesh(core_axis_name="c", subcore_axis_name="s")

    @functools.partial(
        pl.kernel,
        out_type=jax.ShapeDtypeStruct((n_rows, _C), jnp.float32),
        mesh=mesh,
        scratch_types=[
            pltpu.VMEM((nch, ch), jnp.int32),
            pltpu.VMEM((ch, _C), jnp.float32),
            pltpu.SemaphoreType.DMA,
        ],
    )
    def gather_kernel(table_hbm, idx_hbm, out_hbm, idx_v, rows_v, sem):
        wid = lax.axis_index("s") * info.num_cores + lax.axis_index("c")
        base = wid * b_per_w
        pltpu.sync_copy(idx_hbm.at[wid], idx_v)
        for j in range(nch):
            pltpu.async_copy(table_hbm.at[idx_v.at[j]], rows_v, sem).wait()
            pltpu.sync_copy(rows_v, out_hbm.at[pl.ds(base + j * ch, ch)])

    return gather_kernel(table, idx3)


def kernel(x, codebook):
    b, c, t, h, w = x.shape
    z = jnp.transpose(x, (0, 2, 3, 4, 1)).reshape(-1, c)
    zz = jnp.sum(z * z, axis=1, keepdims=True)
    cc = jnp.sum(codebook * codebook, axis=1)
    z_bf = z.astype(jnp.bfloat16)
    cb_bf = codebook.astype(jnp.bfloat16)
    kpad = _NKW * _BK - _K
    cb_pad = jnp.pad(cb_bf, ((0, kpad), (0, 0)))
    cc_pad = jnp.pad(cc, (0, kpad)).reshape(_NKW, 1, _BK)
    idx, loss_sum = _dist_argmin(z_bf, zz, cb_pad, cc_pad)
    q = _sc_gather(codebook, idx)
    n_rows = z.shape[0]
    loss = loss_sum * ((1.0 + _BETA) / (n_rows * c))
    q_st = z + (q - z)
    x_q = jnp.transpose(q_st.reshape(b, t, h, w, c), (0, 4, 1, 2, 3))
    indices = idx.reshape(b, t, h, w)
    return x_q, loss, indices


# speedup vs baseline: 1.0536x; 1.0112x over previous
import functools

import jax
import jax.numpy as jnp
from jax import lax
from jax.experimental import pallas as pl
from jax.experimental.pallas import tpu as pltpu
from jax.experimental.pallas import tpu_sc as plsc

_K = 8192
_C = 256
_BETA = 0.25

_BN = 512
_BK = 2736
_NKW = 3


def _dist_argmin_body(z_ref, zz_ref, cb_ref, cc_ref, idx_ref, lossv_ref,
                      accv_ref, acci_ref, accl_ref):
    k = pl.program_id(1)
    nk = pl.num_programs(1)

    @pl.when(k == 0)
    def _init():
        accv_ref[...] = jnp.full((_BN, 1), jnp.inf, jnp.float32)
        acci_ref[...] = jnp.zeros((_BN, 1), jnp.int32)
        accl_ref[...] = jnp.zeros((_BN, 1), jnp.float32)

    cb = cb_ref[pl.ds(k * _BK, _BK), :]
    dot = lax.dot_general(z_ref[...], cb, (((1,), (1,)), ((), ())),
                          preferred_element_type=jnp.float32)
    d = (zz_ref[...] - 2.0 * dot) + cc_ref[0]
    cols = lax.broadcasted_iota(jnp.int32, (_BN, _BK), 1) + k * _BK
    d = jnp.where(cols < _K, d, jnp.inf)
    wv = jnp.min(d, axis=1, keepdims=True)
    wi = jnp.min(jnp.where(d == wv, cols, _K), axis=1, keepdims=True)
    take = wv < accv_ref[...]
    accv_ref[...] = jnp.where(
        take, wv.astype(jnp.bfloat16).astype(jnp.float32), accv_ref[...])
    acci_ref[...] = jnp.where(take, wi, acci_ref[...])
    accl_ref[...] = jnp.where(take, wv, accl_ref[...])

    @pl.when(k == nk - 1)
    def _finalize():
        idx_ref[...] = acci_ref[...]
        lossv_ref[...] = accl_ref[...]


def _dist_argmin(z_bf, zz, cb_bf, cc):
    n_rows = z_bf.shape[0]
    grid = (n_rows // _BN, _NKW)
    idx2, lossv = pl.pallas_call(
        _dist_argmin_body,
        grid=grid,
        in_specs=[
            pl.BlockSpec((_BN, _C), lambda n, k: (n, 0)),
            pl.BlockSpec((_BN, 1), lambda n, k: (n, 0)),
            pl.BlockSpec((_NKW * _BK, _C), lambda n, k: (0, 0)),
            pl.BlockSpec((1, 1, _BK), lambda n, k: (k, 0, 0)),
        ],
        out_specs=[
            pl.BlockSpec((_BN, 1), lambda n, k: (n, 0)),
            pl.BlockSpec((_BN, 1), lambda n, k: (n, 0)),
        ],
        out_shape=[
            jax.ShapeDtypeStruct((n_rows, 1), jnp.int32),
            jax.ShapeDtypeStruct((n_rows, 1), jnp.float32),
        ],
        scratch_shapes=[
            pltpu.VMEM((_BN, 1), jnp.float32),
            pltpu.VMEM((_BN, 1), jnp.int32),
            pltpu.VMEM((_BN, 1), jnp.float32),
        ],
        compiler_params=pltpu.CompilerParams(
            dimension_semantics=("parallel", "arbitrary")),
    )(z_bf, zz, cb_bf, cc)
    return idx2.reshape(-1), jnp.sum(lossv)


def _sc_gather(table, idx):
    info = plsc.get_sparse_core_info()
    nw = info.num_cores * info.num_subcores
    n_rows = idx.shape[0]
    ch = 128
    b_per_w = n_rows // nw
    nch = b_per_w // ch
    idx3 = idx.reshape(nw, nch, ch)
    mesh = plsc.VectorSubcoreMesh(core_axis_name="c", subcore_axis_name="s")

    @functools.partial(
        pl.kernel,
        out_type=jax.ShapeDtypeStruct((n_rows, _C), jnp.float32),
        mesh=mesh,
        scratch_types=[
            pltpu.VMEM((nch, ch), jnp.int32),
            pltpu.VMEM((ch, _C), jnp.float32),
            pltpu.SemaphoreType.DMA,
        ],
    )
    def gather_kernel(table_hbm, idx_hbm, out_hbm, idx_v, rows_v, sem):
        wid = lax.axis_index("s") * info.num_cores + lax.axis_index("c")
        base = wid * b_per_w
        pltpu.sync_copy(idx_hbm.at[wid], idx_v)
        for j in range(nch):
            pltpu.async_copy(table_hbm.at[idx_v.at[j]], rows_v, sem).wait()
            pltpu.sync_copy(rows_v, out_hbm.at[pl.ds(base + j * ch, ch)])

    return gather_kernel(table, idx3)


def kernel(x, codebook):
    b, c, t, h, w = x.shape
    z = jnp.transpose(x, (0, 2, 3, 4, 1)).reshape(-1, c)
    zz = jnp.sum(z * z, axis=1, keepdims=True)
    cc = jnp.sum(codebook * codebook, axis=1)
    z_bf = z.astype(jnp.bfloat16)
    cb_bf = codebook.astype(jnp.bfloat16)
    kpad = _NKW * _BK - _K
    cb_pad = jnp.pad(cb_bf, ((0, kpad), (0, 0)))
    cc_pad = jnp.pad(cc, (0, kpad)).reshape(_NKW, 1, _BK)
    idx, loss_sum = _dist_argmin(z_bf, zz, cb_pad, cc_pad)
    q = _sc_gather(codebook, idx)
    n_rows = z.shape[0]
    loss = loss_sum * ((1.0 + _BETA) / (n_rows * c))
    q_st = z + (q - z)
    x_q = jnp.transpose(q_st.reshape(b, t, h, w, c), (0, 4, 1, 2, 3))
    indices = idx.reshape(b, t, h, w)
    return x_q, loss, indices


# speedup vs baseline: 1.2162x; 1.1543x over previous
import functools

import jax
import jax.numpy as jnp
from jax import lax
from jax.experimental import pallas as pl
from jax.experimental.pallas import tpu as pltpu
from jax.experimental.pallas import tpu_sc as plsc

_K = 8192
_C = 256
_BETA = 0.25

_BN = 512
_BK = 2736
_NKW = 3


def _dist_argmin_body(z_ref, zz_ref, cb_ref, cc_ref, colf_ref, idx_ref,
                      lossv_ref, accv_ref, acci_ref, accl_ref):
    k = pl.program_id(1)
    nk = pl.num_programs(1)

    @pl.when(k == 0)
    def _init():
        accv_ref[...] = jnp.full((_BN, 1), jnp.inf, jnp.float32)
        acci_ref[...] = jnp.zeros((_BN, 1), jnp.int32)
        accl_ref[...] = jnp.zeros((_BN, 1), jnp.float32)

    cb = cb_ref[pl.ds(k * _BK, _BK), :]
    dot = lax.dot_general(z_ref[...], cb, (((1,), (1,)), ((), ())),
                          preferred_element_type=jnp.float32)
    d = (zz_ref[...] - 2.0 * dot) + cc_ref[0]
    wv = jnp.min(d, axis=1, keepdims=True)
    wi = jnp.min(jnp.where(d == wv, colf_ref[0], jnp.float32(_K)),
                 axis=1, keepdims=True)
    take = wv < accv_ref[...]
    accv_ref[...] = jnp.where(
        take, wv.astype(jnp.bfloat16).astype(jnp.float32), accv_ref[...])
    acci_ref[...] = jnp.where(take, wi.astype(jnp.int32), acci_ref[...])
    accl_ref[...] = jnp.where(take, wv, accl_ref[...])

    @pl.when(k == nk - 1)
    def _finalize():
        idx_ref[...] = acci_ref[...]
        lossv_ref[...] = accl_ref[...]


def _dist_argmin(z_bf, zz, cb_bf, cc, colf):
    n_rows = z_bf.shape[0]
    grid = (n_rows // _BN, _NKW)
    idx2, lossv = pl.pallas_call(
        _dist_argmin_body,
        grid=grid,
        in_specs=[
            pl.BlockSpec((_BN, _C), lambda n, k: (n, 0)),
            pl.BlockSpec((_BN, 1), lambda n, k: (n, 0)),
            pl.BlockSpec((_NKW * _BK, _C), lambda n, k: (0, 0)),
            pl.BlockSpec((1, 1, _BK), lambda n, k: (k, 0, 0)),
            pl.BlockSpec((1, 1, _BK), lambda n, k: (k, 0, 0)),
        ],
        out_specs=[
            pl.BlockSpec((_BN, 1), lambda n, k: (n, 0)),
            pl.BlockSpec((_BN, 1), lambda n, k: (n, 0)),
        ],
        out_shape=[
            jax.ShapeDtypeStruct((n_rows, 1), jnp.int32),
            jax.ShapeDtypeStruct((n_rows, 1), jnp.float32),
        ],
        scratch_shapes=[
            pltpu.VMEM((_BN, 1), jnp.float32),
            pltpu.VMEM((_BN, 1), jnp.int32),
            pltpu.VMEM((_BN, 1), jnp.float32),
        ],
        compiler_params=pltpu.CompilerParams(
            dimension_semantics=("parallel", "arbitrary")),
    )(z_bf, zz, cb_bf, cc, colf)
    return idx2.reshape(-1), jnp.sum(lossv)


def _sc_gather(table, idx):
    info = plsc.get_sparse_core_info()
    nw = info.num_cores * info.num_subcores
    n_rows = idx.shape[0]
    ch = 128
    b_per_w = n_rows // nw
    nch = b_per_w // ch
    idx3 = idx.reshape(nw, nch, ch)
    mesh = plsc.VectorSubcoreMesh(core_axis_name="c", subcore_axis_name="s")

    @functools.partial(
        pl.kernel,
        out_type=jax.ShapeDtypeStruct((n_rows, _C), jnp.float32),
        mesh=mesh,
        scratch_types=[
            pltpu.VMEM((nch, ch), jnp.int32),
            pltpu.VMEM((ch, _C), jnp.float32),
            pltpu.SemaphoreType.DMA,
        ],
    )
    def gather_kernel(table_hbm, idx_hbm, out_hbm, idx_v, rows_v, sem):
        wid = lax.axis_index("s") * info.num_cores + lax.axis_index("c")
        base = wid * b_per_w
        pltpu.sync_copy(idx_hbm.at[wid], idx_v)
        for j in range(nch):
            pltpu.async_copy(table_hbm.at[idx_v.at[j]], rows_v, sem).wait()
            pltpu.sync_copy(rows_v, out_hbm.at[pl.ds(base + j * ch, ch)])

    return gather_kernel(table, idx3)


def kernel(x, codebook):
    b, c, t, h, w = x.shape
    z = jnp.transpose(x, (0, 2, 3, 4, 1)).reshape(-1, c)
    zz = jnp.sum(z * z, axis=1, keepdims=True)
    cc = jnp.sum(codebook * codebook, axis=1)
    z_bf = z.astype(jnp.bfloat16)
    cb_bf = codebook.astype(jnp.bfloat16)
    kpad = _NKW * _BK - _K
    cb_pad = jnp.pad(cb_bf, ((0, kpad), (0, 0)))
    cc_pad = jnp.pad(cc, (0, kpad),
                     constant_values=jnp.inf).reshape(_NKW, 1, _BK)
    colf = jnp.arange(_NKW * _BK, dtype=jnp.float32).reshape(_NKW, 1, _BK)
    idx, loss_sum = _dist_argmin(z_bf, zz, cb_pad, cc_pad, colf)
    q = _sc_gather(codebook, idx)
    n_rows = z.shape[0]
    loss = loss_sum * ((1.0 + _BETA) / (n_rows * c))
    q_st = z + (q - z)
    x_q = jnp.transpose(q_st.reshape(b, t, h, w, c), (0, 4, 1, 2, 3))
    indices = idx.reshape(b, t, h, w)
    return x_q, loss, indices


# speedup vs baseline: 1.3631x; 1.1208x over previous
import functools

import jax
import jax.numpy as jnp
from jax import lax
from jax.experimental import pallas as pl
from jax.experimental.pallas import tpu as pltpu
from jax.experimental.pallas import tpu_sc as plsc

_K = 8192
_C = 256
_BETA = 0.25

_BN = 2048
_BK = 2736
_NKW = 3


def _dist_argmin_body(z_ref, zz_ref, cb_ref, cc_ref, colf_ref, idx_ref,
                      lossv_ref, accv_ref, acci_ref, accl_ref):
    k = pl.program_id(1)
    nk = pl.num_programs(1)

    @pl.when(k == 0)
    def _init():
        accv_ref[...] = jnp.full((_BN, 1), jnp.inf, jnp.float32)
        acci_ref[...] = jnp.zeros((_BN, 1), jnp.int32)
        accl_ref[...] = jnp.zeros((_BN, 1), jnp.float32)

    cb = cb_ref[pl.ds(k * _BK, _BK), :]
    dot = lax.dot_general(z_ref[...], cb, (((1,), (1,)), ((), ())),
                          preferred_element_type=jnp.float32)
    d = (zz_ref[...] - 2.0 * dot) + cc_ref[0]
    wv = jnp.min(d, axis=1, keepdims=True)
    wi = jnp.min(jnp.where(d == wv, colf_ref[0], jnp.float32(_K)),
                 axis=1, keepdims=True)
    take = wv < accv_ref[...]
    accv_ref[...] = jnp.where(
        take, wv.astype(jnp.bfloat16).astype(jnp.float32), accv_ref[...])
    acci_ref[...] = jnp.where(take, wi.astype(jnp.int32), acci_ref[...])
    accl_ref[...] = jnp.where(take, wv, accl_ref[...])

    @pl.when(k == nk - 1)
    def _finalize():
        idx_ref[...] = acci_ref[...]
        lossv_ref[...] = accl_ref[...]


def _dist_argmin(z_bf, zz, cb_bf, cc, colf):
    n_rows = z_bf.shape[0]
    grid = (n_rows // _BN, _NKW)
    idx2, lossv = pl.pallas_call(
        _dist_argmin_body,
        grid=grid,
        in_specs=[
            pl.BlockSpec((_BN, _C), lambda n, k: (n, 0)),
            pl.BlockSpec((_BN, 1), lambda n, k: (n, 0)),
            pl.BlockSpec((_NKW * _BK, _C), lambda n, k: (0, 0)),
            pl.BlockSpec((1, 1, _BK), lambda n, k: (k, 0, 0)),
            pl.BlockSpec((1, 1, _BK), lambda n, k: (k, 0, 0)),
        ],
        out_specs=[
            pl.BlockSpec((_BN, 1), lambda n, k: (n, 0)),
            pl.BlockSpec((_BN, 1), lambda n, k: (n, 0)),
        ],
        out_shape=[
            jax.ShapeDtypeStruct((n_rows, 1), jnp.int32),
            jax.ShapeDtypeStruct((n_rows, 1), jnp.float32),
        ],
        scratch_shapes=[
            pltpu.VMEM((_BN, 1), jnp.float32),
            pltpu.VMEM((_BN, 1), jnp.int32),
            pltpu.VMEM((_BN, 1), jnp.float32),
        ],
        compiler_params=pltpu.CompilerParams(
            dimension_semantics=("parallel", "arbitrary")),
    )(z_bf, zz, cb_bf, cc, colf)
    return idx2.reshape(-1), jnp.sum(lossv)


def _sc_gather(table, idx):
    info = plsc.get_sparse_core_info()
    nw = info.num_cores * info.num_subcores
    n_rows = idx.shape[0]
    ch = 128
    b_per_w = n_rows // nw
    nch = b_per_w // ch
    idx3 = idx.reshape(nw, nch, ch)
    mesh = plsc.VectorSubcoreMesh(core_axis_name="c", subcore_axis_name="s")

    @functools.partial(
        pl.kernel,
        out_type=jax.ShapeDtypeStruct((n_rows, _C), jnp.float32),
        mesh=mesh,
        scratch_types=[
            pltpu.VMEM((nch, ch), jnp.int32),
            pltpu.VMEM((ch, _C), jnp.float32),
            pltpu.SemaphoreType.DMA,
        ],
    )
    def gather_kernel(table_hbm, idx_hbm, out_hbm, idx_v, rows_v, sem):
        wid = lax.axis_index("s") * info.num_cores + lax.axis_index("c")
        base = wid * b_per_w
        pltpu.sync_copy(idx_hbm.at[wid], idx_v)
        for j in range(nch):
            pltpu.async_copy(table_hbm.at[idx_v.at[j]], rows_v, sem).wait()
            pltpu.sync_copy(rows_v, out_hbm.at[pl.ds(base + j * ch, ch)])

    return gather_kernel(table, idx3)


def kernel(x, codebook):
    b, c, t, h, w = x.shape
    z = jnp.transpose(x, (0, 2, 3, 4, 1)).reshape(-1, c)
    zz = jnp.sum(z * z, axis=1, keepdims=True)
    cc = jnp.sum(codebook * codebook, axis=1)
    z_bf = z.astype(jnp.bfloat16)
    cb_bf = codebook.astype(jnp.bfloat16)
    kpad = _NKW * _BK - _K
    cb_pad = jnp.pad(cb_bf, ((0, kpad), (0, 0)))
    cc_pad = jnp.pad(cc, (0, kpad),
                     constant_values=jnp.inf).reshape(_NKW, 1, _BK)
    colf = jnp.arange(_NKW * _BK, dtype=jnp.float32).reshape(_NKW, 1, _BK)
    idx, loss_sum = _dist_argmin(z_bf, zz, cb_pad, cc_pad, colf)
    q = _sc_gather(codebook, idx)
    n_rows = z.shape[0]
    loss = loss_sum * ((1.0 + _BETA) / (n_rows * c))
    q_st = z + (q - z)
    x_q = jnp.transpose(q_st.reshape(b, t, h, w, c), (0, 4, 1, 2, 3))
    indices = idx.reshape(b, t, h, w)
    return x_q, loss, indices


# speedup vs baseline: 1.3686x; 1.0041x over previous
import functools

import jax
import jax.numpy as jnp
from jax import lax
from jax.experimental import pallas as pl
from jax.experimental.pallas import tpu as pltpu
from jax.experimental.pallas import tpu_sc as plsc

_K = 8192
_C = 256
_BETA = 0.25

_BN = 2048
_BK = 2736
_NKW = 3


def _dist_argmin_body(z_ref, zz_ref, cb_ref, cc_ref, colf_ref, idx_ref,
                      lossv_ref, accv_ref, acci_ref, accl_ref):
    k = pl.program_id(1)
    nk = pl.num_programs(1)

    @pl.when(k == 0)
    def _init():
        accv_ref[...] = jnp.full((_BN, 1), jnp.inf, jnp.float32)
        acci_ref[...] = jnp.zeros((_BN, 1), jnp.int32)
        accl_ref[...] = jnp.zeros((_BN, 1), jnp.float32)

    cb = cb_ref[pl.ds(k * _BK, _BK), :]
    dot = lax.dot_general(z_ref[...], cb, (((1,), (1,)), ((), ())),
                          preferred_element_type=jnp.float32)
    d = (zz_ref[...] - 2.0 * dot) + cc_ref[0]
    wv = jnp.min(d, axis=1, keepdims=True)
    wi = jnp.min(jnp.where(d == wv, colf_ref[0], jnp.float32(_K)),
                 axis=1, keepdims=True)
    take = wv < accv_ref[...]
    accv_ref[...] = jnp.where(
        take, wv.astype(jnp.bfloat16).astype(jnp.float32), accv_ref[...])
    acci_ref[...] = jnp.where(take, wi.astype(jnp.int32), acci_ref[...])
    accl_ref[...] = jnp.where(take, wv, accl_ref[...])

    @pl.when(k == nk - 1)
    def _finalize():
        idx_ref[...] = acci_ref[...]
        lossv_ref[...] = accl_ref[...]


def _dist_argmin(z_bf, zz, cb_bf, cc, colf):
    n_rows = z_bf.shape[0]
    grid = (n_rows // _BN, _NKW)
    idx2, lossv = pl.pallas_call(
        _dist_argmin_body,
        grid=grid,
        in_specs=[
            pl.BlockSpec((_BN, _C), lambda n, k: (n, 0)),
            pl.BlockSpec((_BN, 1), lambda n, k: (n, 0)),
            pl.BlockSpec((_NKW * _BK, _C), lambda n, k: (0, 0)),
            pl.BlockSpec((1, 1, _BK), lambda n, k: (k, 0, 0)),
            pl.BlockSpec((1, 1, _BK), lambda n, k: (k, 0, 0)),
        ],
        out_specs=[
            pl.BlockSpec((_BN, 1), lambda n, k: (n, 0)),
            pl.BlockSpec((_BN, 1), lambda n, k: (n, 0)),
        ],
        out_shape=[
            jax.ShapeDtypeStruct((n_rows, 1), jnp.int32),
            jax.ShapeDtypeStruct((n_rows, 1), jnp.float32),
        ],
        scratch_shapes=[
            pltpu.VMEM((_BN, 1), jnp.float32),
            pltpu.VMEM((_BN, 1), jnp.int32),
            pltpu.VMEM((_BN, 1), jnp.float32),
        ],
        compiler_params=pltpu.CompilerParams(
            dimension_semantics=("parallel", "arbitrary")),
    )(z_bf, zz, cb_bf, cc, colf)
    return idx2.reshape(-1), jnp.sum(lossv)


def _sc_gather(table, idx):
    info = plsc.get_sparse_core_info()
    nw = info.num_cores * info.num_subcores
    n_rows = idx.shape[0]
    ch = 128
    b_per_w = n_rows // nw
    nch = b_per_w // ch
    idx3 = idx.reshape(nw, nch, ch)
    mesh = plsc.VectorSubcoreMesh(core_axis_name="c", subcore_axis_name="s")

    @functools.partial(
        pl.kernel,
        out_type=jax.ShapeDtypeStruct((n_rows, _C), jnp.float32),
        mesh=mesh,
        scratch_types=[
            pltpu.VMEM((nch, ch), jnp.int32),
            pltpu.VMEM((2, ch, _C), jnp.float32),
            pltpu.SemaphoreType.DMA,
            pltpu.SemaphoreType.DMA,
        ],
    )
    def gather_kernel(table_hbm, idx_hbm, out_hbm, idx_v, rows_v, sem_a,
                      sem_b):
        wid = lax.axis_index("s") * info.num_cores + lax.axis_index("c")
        base = wid * b_per_w
        pltpu.sync_copy(idx_hbm.at[wid], idx_v)
        sems = (sem_a, sem_b)
        cps = [pltpu.async_copy(table_hbm.at[idx_v.at[j]], rows_v.at[j % 2],
                                sems[j % 2])
               for j in range(min(2, nch))]
        for j in range(nch):
            cps[j].wait()
            pltpu.sync_copy(rows_v.at[j % 2],
                            out_hbm.at[pl.ds(base + j * ch, ch)])
            if j + 2 < nch:
                cps.append(pltpu.async_copy(table_hbm.at[idx_v.at[j + 2]],
                                            rows_v.at[j % 2], sems[j % 2]))

    return gather_kernel(table, idx3)


def kernel(x, codebook):
    b, c, t, h, w = x.shape
    z = jnp.transpose(x, (0, 2, 3, 4, 1)).reshape(-1, c)
    zz = jnp.sum(z * z, axis=1, keepdims=True)
    cc = jnp.sum(codebook * codebook, axis=1)
    z_bf = z.astype(jnp.bfloat16)
    cb_bf = codebook.astype(jnp.bfloat16)
    kpad = _NKW * _BK - _K
    cb_pad = jnp.pad(cb_bf, ((0, kpad), (0, 0)))
    cc_pad = jnp.pad(cc, (0, kpad),
                     constant_values=jnp.inf).reshape(_NKW, 1, _BK)
    colf = jnp.arange(_NKW * _BK, dtype=jnp.float32).reshape(_NKW, 1, _BK)
    idx, loss_sum = _dist_argmin(z_bf, zz, cb_pad, cc_pad, colf)
    q = _sc_gather(codebook, idx)
    n_rows = z.shape[0]
    loss = loss_sum * ((1.0 + _BETA) / (n_rows * c))
    q_st = z + (q - z)
    x_q = jnp.transpose(q_st.reshape(b, t, h, w, c), (0, 4, 1, 2, 3))
    indices = idx.reshape(b, t, h, w)
    return x_q, loss, indices
